# baseline (speedup 1.0000x reference)
import jax
import jax.numpy as jnp
from jax import lax
from jax.experimental import pallas as pl
from jax.experimental.pallas import tpu as pltpu
from jax.experimental.pallas import tpu_sc as plsc

VOCAB1 = 100001
EMBED = 128
HIDDEN = 512
OUT = 128
B = 4096
L = 50
LP = 64

NC = 2
NS = 16
NW = NC * NS
ROWS_PER_W = B // NW
RB = 2
NSTREAM = ROWS_PER_W // RB
NBUF = 2
NLG = EMBED // 16


def _sc_pool_body(x_r, table, out_hbm, idx_v, buf0, buf1, out_v, sem0, sem1):
    bufs = (buf0, buf1)
    sems = (sem0, sem1)
    wid = lax.axis_index("s") * NC + lax.axis_index("c")

    pltpu.sync_copy(x_r.at[wid], idx_v)

    def gather(s, k):
        return pltpu.async_copy(table.at[idx_v.at[s]], bufs[k], sems[k])

    for k in range(NBUF):
        gather(k, k)

    def accum(buf, s):
        for rp in range(RB // 2):
            def jbody(j, accs, rp=rp):
                new0 = tuple(
                    accs[c] + buf[2 * rp * LP + j, pl.ds(16 * c, 16)]
                    for c in range(NLG))
                new1 = tuple(
                    accs[NLG + c] + buf[(2 * rp + 1) * LP + j,
                                        pl.ds(16 * c, 16)]
                    for c in range(NLG))
                return new0 + new1

            init = tuple(jnp.zeros((16,), jnp.float32)
                         for _ in range(2 * NLG))
            accs = lax.fori_loop(0, LP, jbody, init)
            row = RB * s + 2 * rp
            for c in range(NLG):
                out_v[row, pl.ds(16 * c, 16)] = accs[c]
                out_v[row + 1, pl.ds(16 * c, 16)] = accs[NLG + c]

    def group_body(g, _):
        for k in range(NBUF):
            s = NBUF * g + k
            pltpu.make_async_copy(
                table.at[idx_v.at[s]], bufs[k], sems[k]).wait()
            accum(bufs[k], s)

            @pl.when(g < NSTREAM // NBUF - 1)
            def _():
                gather(s + NBUF, k)

        return 0

    lax.fori_loop(0, NSTREAM // NBUF, group_body, 0)

    pltpu.sync_copy(out_v, out_hbm.at[pl.ds(wid * ROWS_PER_W, ROWS_PER_W)])


def _sc_pool(x_r, table):
    mesh = plsc.VectorSubcoreMesh(core_axis_name="c", subcore_axis_name="s")
    return pl.kernel(
        _sc_pool_body,
        out_type=jax.ShapeDtypeStruct((B, EMBED), jnp.float32),
        mesh=mesh,
        scratch_types=[
            pltpu.VMEM((NSTREAM, RB * LP), jnp.int32),
            pltpu.VMEM((RB * LP, EMBED), jnp.float32),
            pltpu.VMEM((RB * LP, EMBED), jnp.float32),
            pltpu.VMEM((ROWS_PER_W, EMBED), jnp.float32),
            pltpu.SemaphoreType.DMA,
            pltpu.SemaphoreType.DMA,
        ],
    )(x_r, table)


BM = 512


def _mlp_body(p_ref, w1_ref, b1_ref, w2_ref, b2_ref, o_ref):
    h = jnp.dot(p_ref[...] * (1.0 / L), w1_ref[...],
                preferred_element_type=jnp.float32)
    h = jnp.maximum(h + b1_ref[...], 0.0)
    o_ref[...] = jnp.dot(h, w2_ref[...],
                         preferred_element_type=jnp.float32) + b2_ref[...]


def _mlp(pooled_sum, W1, b1, W2, b2):
    return pl.pallas_call(
        _mlp_body,
        grid=(B // BM,),
        in_specs=[
            pl.BlockSpec((BM, EMBED), lambda i: (i, 0)),
            pl.BlockSpec((EMBED, HIDDEN), lambda i: (0, 0)),
            pl.BlockSpec((1, HIDDEN), lambda i: (0, 0)),
            pl.BlockSpec((HIDDEN, OUT), lambda i: (0, 0)),
            pl.BlockSpec((1, OUT), lambda i: (0, 0)),
        ],
        out_specs=pl.BlockSpec((BM, OUT), lambda i: (i, 0)),
        out_shape=jax.ShapeDtypeStruct((B, OUT), jnp.float32),
    )(pooled_sum, W1, b1.reshape(1, HIDDEN), W2, b2.reshape(1, OUT))


@jax.jit
def kernel(x, table, W1, b1, W2, b2):
    xp = jnp.pad(x.astype(jnp.int32), ((0, 0), (0, LP - L)))
    x_r = xp.reshape(NW, NSTREAM, RB * LP)
    pooled_sum = _sc_pool(x_r, table)
    return _mlp(pooled_sum, W1, b1, W2, b2)


# speedup vs baseline: 25.5748x; 25.5748x over previous
import jax
import jax.numpy as jnp
from jax import lax
from jax.experimental import pallas as pl
from jax.experimental.pallas import tpu as pltpu
from jax.experimental.pallas import tpu_sc as plsc

VOCAB1 = 100001
EMBED = 128
HIDDEN = 512
OUT = 128
B = 4096
L = 50

NC = 2
NS = 16
NW = NC * NS
ROWS_PER_W = B // NW
RB = 2
CNT = RB * L
CNTP = 104
NSTREAM = ROWS_PER_W // RB
NBUF = 2
NLG = EMBED // 16


def _sc_pool_body(x_r, table, out_hbm, idx_v, buf0, buf1, out_v, sem0, sem1):
    bufs = (buf0, buf1)
    sems = (sem0, sem1)
    wid = lax.axis_index("s") * NC + lax.axis_index("c")

    pltpu.sync_copy(x_r.at[wid], idx_v)

    def gather(s, k):
        return pltpu.async_copy(table.at[idx_v.at[s]], bufs[k], sems[k])

    for k in range(NBUF):
        gather(k, k)

    def accum(buf, s):
        def jbody(j, accs):
            new0 = tuple(accs[c] + buf[j, pl.ds(16 * c, 16)]
                         for c in range(NLG))
            new1 = tuple(accs[NLG + c] + buf[j + L, pl.ds(16 * c, 16)]
                         for c in range(NLG))
            return new0 + new1

        init = tuple(jnp.zeros((16,), jnp.float32) for _ in range(2 * NLG))
        accs = lax.fori_loop(0, L, jbody, init)
        row = RB * s
        for c in range(NLG):
            out_v[row, pl.ds(16 * c, 16)] = accs[c]
            out_v[row + 1, pl.ds(16 * c, 16)] = accs[NLG + c]

    def group_body(g, _):
        for k in range(NBUF):
            s = NBUF * g + k
            pltpu.make_async_copy(
                table.at[idx_v.at[s]], bufs[k], sems[k]).wait()
            accum(bufs[k], s)

            @pl.when(g < NSTREAM // NBUF - 1)
            def _():
                gather(s + NBUF, k)

        return 0

    lax.fori_loop(0, NSTREAM // NBUF, group_body, 0)

    pltpu.sync_copy(out_v, out_hbm.at[pl.ds(wid * ROWS_PER_W, ROWS_PER_W)])


def _sc_pool(x_r, table):
    mesh = plsc.VectorSubcoreMesh(core_axis_name="c", subcore_axis_name="s")
    return pl.kernel(
        _sc_pool_body,
        out_type=jax.ShapeDtypeStruct((B, EMBED), jnp.float32),
        mesh=mesh,
        scratch_types=[
            pltpu.VMEM((NSTREAM, CNTP), jnp.int32),
            pltpu.VMEM((CNTP, EMBED), jnp.float32),
            pltpu.VMEM((CNTP, EMBED), jnp.float32),
            pltpu.VMEM((ROWS_PER_W, EMBED), jnp.float32),
            pltpu.SemaphoreType.DMA,
            pltpu.SemaphoreType.DMA,
        ],
    )(x_r, table)


BM = 512


def _mlp_body(p_ref, w1_ref, b1_ref, w2_ref, b2_ref, o_ref):
    h = jnp.dot(p_ref[...] * (1.0 / L), w1_ref[...],
                preferred_element_type=jnp.float32)
    h = jnp.maximum(h + b1_ref[...], 0.0)
    o_ref[...] = jnp.dot(h, w2_ref[...],
                         preferred_element_type=jnp.float32) + b2_ref[...]


def _mlp(pooled_sum, W1, b1, W2, b2):
    return pl.pallas_call(
        _mlp_body,
        grid=(B // BM,),
        in_specs=[
            pl.BlockSpec((BM, EMBED), lambda i: (i, 0)),
            pl.BlockSpec((EMBED, HIDDEN), lambda i: (0, 0)),
            pl.BlockSpec((1, HIDDEN), lambda i: (0, 0)),
            pl.BlockSpec((HIDDEN, OUT), lambda i: (0, 0)),
            pl.BlockSpec((1, OUT), lambda i: (0, 0)),
        ],
        out_specs=pl.BlockSpec((BM, OUT), lambda i: (i, 0)),
        out_shape=jax.ShapeDtypeStruct((B, OUT), jnp.float32),
    )(pooled_sum, W1, b1.reshape(1, HIDDEN), W2, b2.reshape(1, OUT))


@jax.jit
def kernel(x, table, W1, b1, W2, b2):
    x100 = x.astype(jnp.int32).reshape(B // RB, CNT)
    pad = jnp.arange(1, (B // RB) * (CNTP - CNT) + 1,
                     dtype=jnp.int32).reshape(B // RB, CNTP - CNT)
    x_r = jnp.concatenate([x100, pad], axis=1).reshape(NW, NSTREAM, CNTP)
    pooled_sum = _sc_pool(x_r, table)
    return _mlp(pooled_sum, W1, b1, W2, b2)


# speedup vs baseline: 33.0304x; 1.2915x over previous
import jax
import jax.numpy as jnp
from jax import lax
from jax.experimental import pallas as pl
from jax.experimental.pallas import tpu as pltpu
from jax.experimental.pallas import tpu_sc as plsc

VOCAB1 = 100001
EMBED = 128
HIDDEN = 512
OUT = 128
B = 4096
L = 50

NC = 2
NS = 16
NW = NC * NS
ROWS_PER_W = B // NW
RB = 2
CNT = RB * L
CNTP = 104
NSTREAM = ROWS_PER_W // RB
NBUF = 4
NLG = EMBED // 16


def _sc_pool_body(x_r, table, out_hbm, idx_v, buf0, buf1, buf2, buf3, out_v,
                  sem0, sem1, sem2, sem3):
    bufs = (buf0, buf1, buf2, buf3)
    sems = (sem0, sem1, sem2, sem3)
    wid = lax.axis_index("s") * NC + lax.axis_index("c")

    pltpu.sync_copy(x_r.at[wid], idx_v)

    def gather(s, k):
        return pltpu.async_copy(table.at[idx_v.at[s]], bufs[k], sems[k])

    for k in range(NBUF):
        gather(k, k)

    def accum(buf, s):
        def jbody(j, accs):
            new0 = tuple(accs[c] + buf[j, pl.ds(16 * c, 16)]
                         for c in range(NLG))
            new1 = tuple(accs[NLG + c] + buf[j + L, pl.ds(16 * c, 16)]
                         for c in range(NLG))
            return new0 + new1

        init = tuple(jnp.zeros((16,), jnp.float32) for _ in range(2 * NLG))
        accs = lax.fori_loop(0, L, jbody, init)
        row = RB * s
        for c in range(NLG):
            out_v[row, pl.ds(16 * c, 16)] = accs[c]
            out_v[row + 1, pl.ds(16 * c, 16)] = accs[NLG + c]

    def group_body(g, _):
        for k in range(NBUF):
            s = NBUF * g + k
            pltpu.make_async_copy(
                table.at[idx_v.at[s]], bufs[k], sems[k]).wait()
            accum(bufs[k], s)

            @pl.when(g < NSTREAM // NBUF - 1)
            def _():
                gather(s + NBUF, k)

        return 0

    lax.fori_loop(0, NSTREAM // NBUF, group_body, 0)

    pltpu.sync_copy(out_v, out_hbm.at[pl.ds(wid * ROWS_PER_W, ROWS_PER_W)])


def _sc_pool(x_r, table):
    mesh = plsc.VectorSubcoreMesh(core_axis_name="c", subcore_axis_name="s")
    return pl.kernel(
        _sc_pool_body,
        out_type=jax.ShapeDtypeStruct((B, EMBED), jnp.float32),
        mesh=mesh,
        scratch_types=[
            pltpu.VMEM((NSTREAM, CNTP), jnp.int32),
            pltpu.VMEM((CNTP, EMBED), jnp.float32),
            pltpu.VMEM((CNTP, EMBED), jnp.float32),
            pltpu.VMEM((CNTP, EMBED), jnp.float32),
            pltpu.VMEM((CNTP, EMBED), jnp.float32),
            pltpu.VMEM((ROWS_PER_W, EMBED), jnp.float32),
            pltpu.SemaphoreType.DMA,
            pltpu.SemaphoreType.DMA,
            pltpu.SemaphoreType.DMA,
            pltpu.SemaphoreType.DMA,
        ],
    )(x_r, table)


BM = 512


def _mlp_body(p_ref, w1_ref, b1_ref, w2_ref, b2_ref, o_ref):
    h = jnp.dot(p_ref[...] * (1.0 / L), w1_ref[...],
                preferred_element_type=jnp.float32)
    h = jnp.maximum(h + b1_ref[...], 0.0)
    o_ref[...] = jnp.dot(h, w2_ref[...],
                         preferred_element_type=jnp.float32) + b2_ref[...]


def _mlp(pooled_sum, W1, b1, W2, b2):
    return pl.pallas_call(
        _mlp_body,
        grid=(B // BM,),
        in_specs=[
            pl.BlockSpec((BM, EMBED), lambda i: (i, 0)),
            pl.BlockSpec((EMBED, HIDDEN), lambda i: (0, 0)),
            pl.BlockSpec((1, HIDDEN), lambda i: (0, 0)),
            pl.BlockSpec((HIDDEN, OUT), lambda i: (0, 0)),
            pl.BlockSpec((1, OUT), lambda i: (0, 0)),
        ],
        out_specs=pl.BlockSpec((BM, OUT), lambda i: (i, 0)),
        out_shape=jax.ShapeDtypeStruct((B, OUT), jnp.float32),
    )(pooled_sum, W1, b1.reshape(1, HIDDEN), W2, b2.reshape(1, OUT))


@jax.jit
def kernel(x, table, W1, b1, W2, b2):
    x100 = x.astype(jnp.int32).reshape(B // RB, CNT)
    pad = jnp.arange(1, (B // RB) * (CNTP - CNT) + 1,
                     dtype=jnp.int32).reshape(B // RB, CNTP - CNT)
    x_r = jnp.concatenate([x100, pad], axis=1).reshape(NW, NSTREAM, CNTP)
    pooled_sum = _sc_pool(x_r, table)
    return _mlp(pooled_sum, W1, b1, W2, b2)
